# Initial kernel scaffold; baseline (speedup 1.0000x reference)
import jax
import jax.numpy as jnp
from jax.experimental import pallas as pl


def kernel(obj_embs, cls_means, W_cls, b_cls, src_idx, tgt_labels):
    raise NotImplementedError("write your pallas kernel here")


# speedup vs baseline: 31.7079x; 31.7079x over previous
import jax
import jax.numpy as jnp
from jax import lax
from jax.experimental import pallas as pl
from jax.experimental.pallas import tpu as pltpu

_C = 81
_KNOWN = 80
_KB = 5
_MKNN = 5
_DELTA = 0.6
_TAU = 0.1
_ALPHA = 0.9
_CQ = 750
_HI = lax.Precision.HIGHEST


def _nt(a, b):
    return lax.dot_general(a, b, (((1,), (1,)), ((), ())), precision=_HI,
                           preferred_element_type=jnp.float32)


def _nn(a, b):
    return lax.dot_general(a, b, (((1,), (0,)), ((), ())), precision=_HI,
                           preferred_element_type=jnp.float32)


def _tn(a, b):
    return lax.dot_general(a, b, (((0,), (0,)), ((), ())), precision=_HI,
                           preferred_element_type=jnp.float32)


def _normalize(x):
    n = jnp.sqrt(jnp.sum(x * x, axis=1, keepdims=True))
    return x / jnp.maximum(n, 1e-12)


def _k1_body(obj_ref, src_ref, lab_ref, matched_ref, seg_ref, cnt_ref):
    b = pl.program_id(0)
    obj = obj_ref[0]
    src = src_ref[0]
    lab = lab_ref[0]
    nm = src.shape[1]
    q = obj.shape[0]
    oh_src_t = (lax.broadcasted_iota(jnp.int32, (q, nm), 0) == src
                ).astype(jnp.float32)
    matched = _tn(oh_src_t, obj)
    matched_ref[0] = matched
    oh_lab_t = (lax.broadcasted_iota(jnp.int32, (_C, nm), 0) == lab
                ).astype(jnp.float32)
    seg = _nn(oh_lab_t, matched)
    cnt = jnp.sum(oh_lab_t, axis=1, keepdims=True)

    @pl.when(b == 0)
    def _():
        seg_ref[...] = jnp.zeros_like(seg_ref)
        cnt_ref[...] = jnp.zeros_like(cnt_ref)

    seg_ref[...] += seg
    cnt_ref[...] += cnt


def _k2_body(cm_ref, seg_ref, cnt_ref, protos_ref, pn_ref, negp_ref):
    cm = cm_ref[...]
    seg = seg_ref[...]
    cnt = cnt_ref[...]
    means = seg / jnp.maximum(cnt, 1.0)
    upd = _ALPHA * cm + (1.0 - _ALPHA) * means
    upd = _normalize(upd)
    protos = jnp.where(cnt > 0.0, upd, cm)
    protos_ref[...] = protos
    pn = _normalize(protos)
    pn_ref[...] = pn
    pmat = _nt(pn, pn) / _TAU
    expp = jnp.exp(pmat)
    eye = (lax.broadcasted_iota(jnp.int32, (_C, _C), 0)
           == lax.broadcasted_iota(jnp.int32, (_C, _C), 1))
    negp = (jnp.sum(expp, axis=1, keepdims=True)
            - jnp.sum(jnp.where(eye, expp, 0.0), axis=1, keepdims=True))
    negp_ref[...] = negp


def _k3_body(obj_ref, matched_ref, srcc_ref, lab_ref, protos_ref, pn_ref,
             w_ref, bias_ref, num_ref, den_ref, cols_ref, segpos_ref,
             pos_ref, simbuf_ref):
    b = pl.program_id(0)
    mq = matched_ref[0]
    srcc = srcc_ref[0]
    lab = lab_ref[0]
    protos = protos_ref[...]
    pn = pn_ref[...]
    wcls = w_ref[...]
    bias = bias_ref[...]
    nm = mq.shape[0]
    q = obj_ref.shape[1]

    mn = _normalize(mq)

    oh_lab_t = (lax.broadcasted_iota(jnp.int32, (_C, nm), 0) == lab
                ).astype(jnp.float32)
    pg = _tn(oh_lab_t, protos)
    diff = mq - pg
    d2_col = jnp.sum(diff * diff, axis=1, keepdims=True)
    d2_row = jnp.sum(diff * diff, axis=1)[None, :]
    m = jnp.where(oh_lab_t > 0, d2_row, -1.0)
    t5 = None
    for _ in range(_KB):
        t5 = jnp.max(m, axis=1, keepdims=True)
        m = jnp.where(m == t5, -1.0, m)
    t5_own = _tn(oh_lab_t, t5)
    sel = d2_col >= t5_own

    nc = q // _CQ
    for c in range(nc):
        c0 = c * _CQ
        objc = obj_ref[0, pl.ds(c0, _CQ), :]
        allnc = _normalize(objc)
        simc = _nt(mn, allnc)
        ismc = jnp.any(
            lax.broadcasted_iota(jnp.int32, (nm, _CQ), 1) == (srcc - c0),
            axis=0)[None, :]
        simbuf_ref[:, pl.ds(c0, _CQ)] = jnp.where(ismc, -4.0, simc)

    t5s = None
    for _ in range(_MKNN):
        cmax = None
        for c in range(nc):
            sc = simbuf_ref[:, pl.ds(c * _CQ, _CQ)]
            if t5s is not None:
                sc = jnp.where(sc < t5s, sc, -4.0)
            part = jnp.max(sc, axis=1, keepdims=True)
            cmax = part if cmax is None else jnp.maximum(cmax, part)
        t5s = cmax

    nv = jnp.zeros((nm, 1), jnp.float32)
    neigh = jnp.zeros_like(mq)
    for c in range(nc):
        c0 = c * _CQ
        sc = simbuf_ref[:, pl.ds(c0, _CQ)]
        wc = ((sc > _DELTA) & (sc >= t5s)).astype(jnp.float32)
        nv = nv + jnp.sum(wc, axis=1, keepdims=True)
        neigh = neigh + _nn(wc, obj_ref[0, pl.ds(c0, _CQ), :])
    g = (mq + neigh) / (1.0 + nv)

    logits = _nt(g, wcls) + bias
    mx = jnp.max(logits, axis=1, keepdims=True)
    lse = jnp.log(jnp.sum(jnp.exp(logits - mx), axis=1, keepdims=True)) + mx
    lastcol = (lax.broadcasted_iota(jnp.int32, (nm, _C), 1) == (_C - 1))
    ce = lse - jnp.sum(jnp.where(lastcol, logits, 0.0), axis=1, keepdims=True)

    valid = (sel & (nv > 0)).astype(jnp.float32)
    num = jnp.sum(ce * valid)
    den = jnp.sum(valid)

    s_t = _nt(pn, mn) / _TAU
    exps = jnp.exp(s_t)
    cols = jnp.sum(exps, axis=1, keepdims=True)
    pos = jnp.sum(jnp.where(oh_lab_t > 0, s_t, 0.0), axis=0)[None, :]
    segpos = jnp.sum(jnp.where(oh_lab_t > 0, exps, 0.0), axis=1,
                     keepdims=True)

    pos_ref[pl.ds(b, 1), :] = pos

    @pl.when(b == 0)
    def _():
        num_ref[...] = jnp.zeros_like(num_ref)
        den_ref[...] = jnp.zeros_like(den_ref)
        cols_ref[...] = jnp.zeros_like(cols_ref)
        segpos_ref[...] = jnp.zeros_like(segpos_ref)

    num_ref[...] += num
    den_ref[...] += den
    cols_ref[...] += cols
    segpos_ref[...] += segpos


def _k4_body(num_ref, den_ref, negp_ref, cols_ref, segpos_ref, pos_ref,
             lab_ref, out_ref):
    num = num_ref[0, 0]
    den = den_ref[0, 0]
    loss_sul = jnp.where(den > 0.0, num / jnp.maximum(den, 1.0), 0.0)

    neg = negp_ref[...] + cols_ref[...] - segpos_ref[...]
    bsz, nm = pos_ref.shape
    total = jnp.float32(0.0)
    for bi in range(bsz):
        lab_b = lab_ref[pl.ds(bi, 1), :]
        pos_b = pos_ref[pl.ds(bi, 1), :]
        oh_t = (lax.broadcasted_iota(jnp.int32, (_C, nm), 0) == lab_b)
        negl = jnp.sum(jnp.where(oh_t, neg, 0.0), axis=0)[None, :]
        e = jnp.exp(pos_b)
        li = -jnp.log(e / (e + negl + 1e-8))
        total = total + jnp.sum(li)
    loss_cec = total / jnp.float32(bsz * nm)

    lane = lax.broadcasted_iota(jnp.int32, (1, 2), 1)
    out_ref[...] = jnp.where(lane == 0, loss_sul, loss_cec)


def kernel(obj_embs, cls_means, W_cls, b_cls, src_idx, tgt_labels):
    bsz, q, d = obj_embs.shape
    nm = src_idx.shape[1]
    src3 = src_idx.reshape(bsz, 1, nm).astype(jnp.int32)
    srcc3 = src_idx.reshape(bsz, nm, 1).astype(jnp.int32)
    lab3 = tgt_labels.reshape(bsz, 1, nm).astype(jnp.int32)
    bias2 = b_cls.reshape(1, _C)
    f32 = jnp.float32

    matched, seg, cnt = pl.pallas_call(
        _k1_body,
        grid=(bsz,),
        in_specs=[
            pl.BlockSpec((1, q, d), lambda b: (b, 0, 0)),
            pl.BlockSpec((1, 1, nm), lambda b: (b, 0, 0)),
            pl.BlockSpec((1, 1, nm), lambda b: (b, 0, 0)),
        ],
        out_specs=[
            pl.BlockSpec((1, nm, d), lambda b: (b, 0, 0)),
            pl.BlockSpec((_C, d), lambda b: (0, 0)),
            pl.BlockSpec((_C, 1), lambda b: (0, 0)),
        ],
        out_shape=[
            jax.ShapeDtypeStruct((bsz, nm, d), f32),
            jax.ShapeDtypeStruct((_C, d), f32),
            jax.ShapeDtypeStruct((_C, 1), f32),
        ],
    )(obj_embs, src3, lab3)

    protos, pn, negp = pl.pallas_call(
        _k2_body,
        out_shape=[
            jax.ShapeDtypeStruct((_C, d), f32),
            jax.ShapeDtypeStruct((_C, d), f32),
            jax.ShapeDtypeStruct((_C, 1), f32),
        ],
    )(cls_means, seg, cnt)

    num, den, cols, segpos, pos = pl.pallas_call(
        _k3_body,
        grid=(bsz,),
        scratch_shapes=[pltpu.VMEM((nm, q), f32)],
        in_specs=[
            pl.BlockSpec((1, q, d), lambda b: (b, 0, 0)),
            pl.BlockSpec((1, nm, d), lambda b: (b, 0, 0)),
            pl.BlockSpec((1, nm, 1), lambda b: (b, 0, 0)),
            pl.BlockSpec((1, 1, nm), lambda b: (b, 0, 0)),
            pl.BlockSpec((_C, d), lambda b: (0, 0)),
            pl.BlockSpec((_C, d), lambda b: (0, 0)),
            pl.BlockSpec((_C, d), lambda b: (0, 0)),
            pl.BlockSpec((1, _C), lambda b: (0, 0)),
        ],
        out_specs=[
            pl.BlockSpec((1, 1), lambda b: (0, 0)),
            pl.BlockSpec((1, 1), lambda b: (0, 0)),
            pl.BlockSpec((_C, 1), lambda b: (0, 0)),
            pl.BlockSpec((_C, 1), lambda b: (0, 0)),
            pl.BlockSpec((bsz, nm), lambda b: (0, 0)),
        ],
        out_shape=[
            jax.ShapeDtypeStruct((1, 1), f32),
            jax.ShapeDtypeStruct((1, 1), f32),
            jax.ShapeDtypeStruct((_C, 1), f32),
            jax.ShapeDtypeStruct((_C, 1), f32),
            jax.ShapeDtypeStruct((bsz, nm), f32),
        ],
    )(obj_embs, matched, srcc3, lab3, protos, pn, W_cls, bias2)

    out = pl.pallas_call(
        _k4_body,
        out_shape=jax.ShapeDtypeStruct((1, 2), f32),
    )(num, den, negp, cols, segpos, pos, tgt_labels.astype(jnp.int32))

    return out.reshape(2)


# speedup vs baseline: 47.0440x; 1.4837x over previous
import jax
import jax.numpy as jnp
from jax import lax
from jax.experimental import pallas as pl
from jax.experimental.pallas import tpu as pltpu

_C = 81
_KNOWN = 80
_KB = 5
_MKNN = 5
_DELTA = 0.6
_TAU = 0.1
_ALPHA = 0.9
_CQ = 750
_HI = lax.Precision.HIGHEST
_MID = lax.Precision.HIGHEST


def _nt(a, b, precision=_HI):
    return lax.dot_general(a, b, (((1,), (1,)), ((), ())), precision=precision,
                           preferred_element_type=jnp.float32)


def _nn(a, b, precision=_HI):
    return lax.dot_general(a, b, (((1,), (0,)), ((), ())), precision=precision,
                           preferred_element_type=jnp.float32)


def _tn(a, b, precision=_HI):
    return lax.dot_general(a, b, (((0,), (0,)), ((), ())), precision=precision,
                           preferred_element_type=jnp.float32)


def _normalize(x):
    n = jnp.sqrt(jnp.sum(x * x, axis=1, keepdims=True))
    return x / jnp.maximum(n, 1e-12)


def _k1_body(obj_ref, src_ref, lab_ref, matched_ref, seg_ref, cnt_ref):
    b = pl.program_id(0)
    obj = obj_ref[0]
    src = src_ref[0]
    lab = lab_ref[0]
    nm = src.shape[1]
    q = obj.shape[0]
    oh_src_t = (lax.broadcasted_iota(jnp.int32, (q, nm), 0) == src
                ).astype(jnp.float32)
    matched = _tn(oh_src_t, obj)
    matched_ref[0] = matched
    oh_lab_t = (lax.broadcasted_iota(jnp.int32, (_C, nm), 0) == lab
                ).astype(jnp.float32)
    seg = _nn(oh_lab_t, matched)
    cnt = jnp.sum(oh_lab_t, axis=1, keepdims=True)

    @pl.when(b == 0)
    def _():
        seg_ref[...] = jnp.zeros_like(seg_ref)
        cnt_ref[...] = jnp.zeros_like(cnt_ref)

    seg_ref[...] += seg
    cnt_ref[...] += cnt


def _k2_body(cm_ref, seg_ref, cnt_ref, protos_ref, pn_ref, negp_ref):
    cm = cm_ref[...]
    seg = seg_ref[...]
    cnt = cnt_ref[...]
    means = seg / jnp.maximum(cnt, 1.0)
    upd = _ALPHA * cm + (1.0 - _ALPHA) * means
    upd = _normalize(upd)
    protos = jnp.where(cnt > 0.0, upd, cm)
    protos_ref[...] = protos
    pn = _normalize(protos)
    pn_ref[...] = pn
    pmat = _nt(pn, pn) / _TAU
    expp = jnp.exp(pmat)
    eye = (lax.broadcasted_iota(jnp.int32, (_C, _C), 0)
           == lax.broadcasted_iota(jnp.int32, (_C, _C), 1))
    negp = (jnp.sum(expp, axis=1, keepdims=True)
            - jnp.sum(jnp.where(eye, expp, 0.0), axis=1, keepdims=True))
    negp_ref[...] = negp


def _split2(x):
    hi = x.astype(jnp.bfloat16)
    lo = (x - hi.astype(jnp.float32)).astype(jnp.bfloat16)
    return hi, lo


def _k3_body(obj_ref, matched_ref, src_ref, lab_ref, labc_ref, protos_ref,
             pn_ref, w_ref, bias_ref, num_ref, den_ref, cols_ref, segpos_ref,
             pos_ref, simbuf_ref):
    b = pl.program_id(0)
    mq = matched_ref[0]
    src = src_ref[0]
    lab = lab_ref[0]
    labc = labc_ref[0]
    protos = protos_ref[...]
    pn = pn_ref[...]
    wcls = w_ref[...]
    bias = bias_ref[...]
    nm, d = mq.shape
    q = obj_ref.shape[1]
    f32 = jnp.float32

    mnorm = jnp.sqrt(jnp.sum(mq * mq, axis=1, keepdims=True))
    mn = mq / jnp.maximum(mnorm, 1e-12)
    mn_hi, mn_lo = _split2(mn)

    oh_lab = (lax.broadcasted_iota(jnp.int32, (nm, _C), 1) == labc
              ).astype(f32)
    pg = _nn(oh_lab, protos)
    diff = mq - pg
    d2_col = jnp.sum(diff * diff, axis=1, keepdims=True)
    mT = jnp.where(oh_lab > 0, d2_col, -1.0)
    t5r = None
    for _ in range(_KB):
        mm = mT if t5r is None else jnp.where(mT < t5r, mT, -1.0)
        t5r = jnp.max(mm, axis=0, keepdims=True)
    t5_own = jnp.sum(oh_lab * t5r, axis=1, keepdims=True)
    sel = d2_col >= t5_own

    nc = q // _CQ
    cands = []
    for c in range(nc):
        c0 = c * _CQ
        objc = obj_ref[0, pl.ds(c0, _CQ), :]
        qnorm = jnp.sqrt(jnp.sum(objc * objc, axis=1, keepdims=True))
        allnc = objc / jnp.maximum(qnorm, 1e-12)
        a_hi, a_lo = _split2(allnc)
        simc = (_nt(a_hi, mn_hi, None) + _nt(a_hi, mn_lo, None)
                + _nt(a_lo, mn_hi, None))
        ism = jnp.any(lax.broadcasted_iota(jnp.int32, (_CQ, nm), 0)
                      == (src - c0), axis=1, keepdims=True)
        simc = jnp.where(ism, -4.0, simc)
        simbuf_ref[pl.ds(c0, _CQ), :] = simc
        tloc = None
        for _ in range(_MKNN):
            sc = simc if tloc is None else jnp.where(simc < tloc, simc, -4.0)
            tloc = jnp.max(sc, axis=0, keepdims=True)
            cands.append(tloc)

    cand = jnp.concatenate(cands, axis=0)
    t5s = None
    for _ in range(_MKNN):
        cm = cand if t5s is None else jnp.where(cand < t5s, cand, -4.0)
        t5s = jnp.max(cm, axis=0, keepdims=True)

    nv_row = jnp.zeros((1, nm), f32)
    neigh = jnp.zeros_like(mq)
    for c in range(nc):
        c0 = c * _CQ
        sc = simbuf_ref[pl.ds(c0, _CQ), :]
        wc = ((sc > _DELTA) & (sc >= t5s)).astype(jnp.bfloat16)
        nv_row = nv_row + jnp.sum(wc.astype(f32), axis=0, keepdims=True)
        o_hi, o_lo = _split2(obj_ref[0, pl.ds(c0, _CQ), :])
        neigh = neigh + _tn(wc, o_hi, None) + _tn(wc, o_lo, None)
    eye_nm = (lax.broadcasted_iota(jnp.int32, (nm, nm), 0)
              == lax.broadcasted_iota(jnp.int32, (nm, nm), 1)).astype(f32)
    nv = _nt(eye_nm, nv_row)
    g = (mq + neigh) / (1.0 + nv)

    logits = _nt(g, wcls) + bias
    mx = jnp.max(logits, axis=1, keepdims=True)
    lse = jnp.log(jnp.sum(jnp.exp(logits - mx), axis=1, keepdims=True)) + mx
    lastcol = (lax.broadcasted_iota(jnp.int32, (nm, _C), 1) == (_C - 1))
    ce = lse - jnp.sum(jnp.where(lastcol, logits, 0.0), axis=1, keepdims=True)

    valid = (sel & (nv > 0)).astype(f32)
    num = jnp.sum(ce * valid)
    den = jnp.sum(valid)

    s_t = _nt(pn, mn) / _TAU
    exps = jnp.exp(s_t)
    oh_lab_t = (lax.broadcasted_iota(jnp.int32, (_C, nm), 0) == lab
                ).astype(f32)
    cols = jnp.sum(exps, axis=1, keepdims=True)
    pos_row = jnp.sum(jnp.where(oh_lab_t > 0, s_t, 0.0), axis=0,
                      keepdims=True)
    segpos = jnp.sum(jnp.where(oh_lab_t > 0, exps, 0.0), axis=1,
                     keepdims=True)

    pos_ref[pl.ds(b, 1), :] = pos_row

    @pl.when(b == 0)
    def _():
        num_ref[...] = jnp.zeros_like(num_ref)
        den_ref[...] = jnp.zeros_like(den_ref)
        cols_ref[...] = jnp.zeros_like(cols_ref)
        segpos_ref[...] = jnp.zeros_like(segpos_ref)

    num_ref[...] += num
    den_ref[...] += den
    cols_ref[...] += cols
    segpos_ref[...] += segpos


def _k4_body(num_ref, den_ref, negp_ref, cols_ref, segpos_ref, pos_ref,
             lab_ref, out_ref):
    num = num_ref[0, 0]
    den = den_ref[0, 0]
    loss_sul = jnp.where(den > 0.0, num / jnp.maximum(den, 1.0), 0.0)

    neg = negp_ref[...] + cols_ref[...] - segpos_ref[...]
    bsz = pos_ref.shape[0]
    nm = pos_ref.shape[1]
    total = jnp.float32(0.0)
    for bi in range(bsz):
        lab_b = lab_ref[bi]
        pos_b = pos_ref[pl.ds(bi, 1), :]
        oh_t = (lax.broadcasted_iota(jnp.int32, (_C, nm), 0) == lab_b)
        negl = jnp.sum(jnp.where(oh_t, neg, 0.0), axis=0)[None, :]
        e = jnp.exp(pos_b)
        li = jnp.log((e + negl + 1e-8) / e)
        total = total + jnp.sum(li)
    loss_cec = total / jnp.float32(bsz * nm)

    lane = lax.broadcasted_iota(jnp.int32, (1, 2), 1)
    out_ref[...] = jnp.where(lane == 0, loss_sul, loss_cec)


def kernel(obj_embs, cls_means, W_cls, b_cls, src_idx, tgt_labels):
    bsz, q, d = obj_embs.shape
    nm = src_idx.shape[1]
    src3 = src_idx.reshape(bsz, 1, nm).astype(jnp.int32)
    lab3 = tgt_labels.reshape(bsz, 1, nm).astype(jnp.int32)
    labc3 = tgt_labels.reshape(bsz, nm, 1).astype(jnp.int32)
    bias2 = b_cls.reshape(1, _C)
    f32 = jnp.float32

    matched, seg, cnt = pl.pallas_call(
        _k1_body,
        grid=(bsz,),
        in_specs=[
            pl.BlockSpec((1, q, d), lambda b: (b, 0, 0)),
            pl.BlockSpec((1, 1, nm), lambda b: (b, 0, 0)),
            pl.BlockSpec((1, 1, nm), lambda b: (b, 0, 0)),
        ],
        out_specs=[
            pl.BlockSpec((1, nm, d), lambda b: (b, 0, 0)),
            pl.BlockSpec((_C, d), lambda b: (0, 0)),
            pl.BlockSpec((_C, 1), lambda b: (0, 0)),
        ],
        out_shape=[
            jax.ShapeDtypeStruct((bsz, nm, d), f32),
            jax.ShapeDtypeStruct((_C, d), f32),
            jax.ShapeDtypeStruct((_C, 1), f32),
        ],
    )(obj_embs, src3, lab3)

    protos, pn, negp = pl.pallas_call(
        _k2_body,
        out_shape=[
            jax.ShapeDtypeStruct((_C, d), f32),
            jax.ShapeDtypeStruct((_C, d), f32),
            jax.ShapeDtypeStruct((_C, 1), f32),
        ],
    )(cls_means, seg, cnt)

    num, den, cols, segpos, pos = pl.pallas_call(
        _k3_body,
        grid=(bsz,),
        scratch_shapes=[pltpu.VMEM((q, nm), f32)],
        in_specs=[
            pl.BlockSpec((1, q, d), lambda b: (b, 0, 0)),
            pl.BlockSpec((1, nm, d), lambda b: (b, 0, 0)),
            pl.BlockSpec((1, 1, nm), lambda b: (b, 0, 0)),
            pl.BlockSpec((1, 1, nm), lambda b: (b, 0, 0)),
            pl.BlockSpec((1, nm, 1), lambda b: (b, 0, 0)),
            pl.BlockSpec((_C, d), lambda b: (0, 0)),
            pl.BlockSpec((_C, d), lambda b: (0, 0)),
            pl.BlockSpec((_C, d), lambda b: (0, 0)),
            pl.BlockSpec((1, _C), lambda b: (0, 0)),
        ],
        out_specs=[
            pl.BlockSpec((1, 1), lambda b: (0, 0)),
            pl.BlockSpec((1, 1), lambda b: (0, 0)),
            pl.BlockSpec((_C, 1), lambda b: (0, 0)),
            pl.BlockSpec((_C, 1), lambda b: (0, 0)),
            pl.BlockSpec((bsz, nm), lambda b: (0, 0)),
        ],
        out_shape=[
            jax.ShapeDtypeStruct((1, 1), f32),
            jax.ShapeDtypeStruct((1, 1), f32),
            jax.ShapeDtypeStruct((_C, 1), f32),
            jax.ShapeDtypeStruct((_C, 1), f32),
            jax.ShapeDtypeStruct((bsz, nm), f32),
        ],
    )(obj_embs, matched, src3, lab3, labc3, protos, pn, W_cls, bias2)

    out = pl.pallas_call(
        _k4_body,
        out_shape=jax.ShapeDtypeStruct((1, 2), f32),
    )(num, den, negp, cols, segpos, pos, lab3)

    return out.reshape(2)


# speedup vs baseline: 51.2174x; 1.0887x over previous
import functools

import jax
import jax.numpy as jnp
from jax import lax
from jax.experimental import pallas as pl
from jax.experimental.pallas import tpu as pltpu
from jax.experimental.pallas import tpu_sc as plsc

_C = 81
_KNOWN = 80
_KB = 5
_MKNN = 5
_DELTA = 0.6
_TAU = 0.1
_ALPHA = 0.9
_CQ = 750
_HI = lax.Precision.HIGHEST
_MID = lax.Precision.HIGHEST


def _nt(a, b, precision=_HI):
    return lax.dot_general(a, b, (((1,), (1,)), ((), ())), precision=precision,
                           preferred_element_type=jnp.float32)


def _nn(a, b, precision=_HI):
    return lax.dot_general(a, b, (((1,), (0,)), ((), ())), precision=precision,
                           preferred_element_type=jnp.float32)


def _tn(a, b, precision=_HI):
    return lax.dot_general(a, b, (((0,), (0,)), ((), ())), precision=precision,
                           preferred_element_type=jnp.float32)


def _normalize(x):
    n = jnp.sqrt(jnp.sum(x * x, axis=1, keepdims=True))
    return x / jnp.maximum(n, 1e-12)


_GCH = 80


def _sc_gather(table, idxp, total, d):
    nw = 32
    npw = total // (nw * _GCH)
    mesh = plsc.VectorSubcoreMesh(core_axis_name="c", subcore_axis_name="s")

    @functools.partial(
        pl.kernel, mesh=mesh,
        out_type=jax.ShapeDtypeStruct((total, d), jnp.float32),
        scratch_types=[
            pltpu.VMEM((_GCH,), jnp.int32),
            pltpu.VMEM((_GCH, d), jnp.float32),
            pltpu.SemaphoreType.DMA,
        ],
    )
    def k(table_hbm, idx_hbm, out_hbm, idx_v, rows_v, sem):
        wid = lax.axis_index("s") * 2 + lax.axis_index("c")
        for j in range(npw):
            base = pl.multiple_of((wid * npw + j) * _GCH, _GCH)
            pltpu.sync_copy(idx_hbm.at[pl.ds(base, _GCH)], idx_v)
            pltpu.async_copy(table_hbm.at[idx_v], rows_v, sem).wait()
            pltpu.sync_copy(rows_v, out_hbm.at[pl.ds(base, _GCH)])

    return k(table, idxp)


def _k2_body(matched_ref, lab_ref, cm_ref, protos_ref, pn_ref, negp_ref):
    mflat = matched_ref[...]
    lab = lab_ref[0]
    cm = cm_ref[...]
    ntot = mflat.shape[0]
    oh_t = (lax.broadcasted_iota(jnp.int32, (_C, ntot), 0) == lab
            ).astype(jnp.float32)
    m_hi, m_lo = _split2(mflat)
    oh_b = oh_t.astype(jnp.bfloat16)
    seg = _nn(oh_b, m_hi, None) + _nn(oh_b, m_lo, None)
    cnt = jnp.sum(oh_t, axis=1, keepdims=True)
    means = seg / jnp.maximum(cnt, 1.0)
    upd = _ALPHA * cm + (1.0 - _ALPHA) * means
    upd = _normalize(upd)
    protos = jnp.where(cnt > 0.0, upd, cm)
    protos_ref[...] = protos
    pn = _normalize(protos)
    pn_ref[...] = pn
    pmat = _nt(pn, pn) / _TAU
    expp = jnp.exp(pmat)
    eye = (lax.broadcasted_iota(jnp.int32, (_C, _C), 0)
           == lax.broadcasted_iota(jnp.int32, (_C, _C), 1))
    negp = (jnp.sum(expp, axis=1, keepdims=True)
            - jnp.sum(jnp.where(eye, expp, 0.0), axis=1, keepdims=True))
    negp_ref[...] = negp


def _split2(x):
    hi = x.astype(jnp.bfloat16)
    lo = (x - hi.astype(jnp.float32)).astype(jnp.bfloat16)
    return hi, lo


def _k3_body(obj_ref, matched_ref, src_ref, lab_ref, labc_ref, protos_ref,
             pn_ref, w_ref, bias_ref, num_ref, den_ref, cols_ref, segpos_ref,
             pos_ref, simbuf_ref):
    b = pl.program_id(0)
    mq = matched_ref[0]
    src = src_ref[0]
    lab = lab_ref[0]
    labc = labc_ref[0]
    protos = protos_ref[...]
    pn = pn_ref[...]
    wcls = w_ref[...]
    bias = bias_ref[...]
    nm, d = mq.shape
    q = obj_ref.shape[1]
    f32 = jnp.float32

    mnorm = jnp.sqrt(jnp.sum(mq * mq, axis=1, keepdims=True))
    mn = mq / jnp.maximum(mnorm, 1e-12)
    mn_hi, mn_lo = _split2(mn)

    oh_lab = (lax.broadcasted_iota(jnp.int32, (nm, _C), 1) == labc
              ).astype(f32)
    pg = _nn(oh_lab, protos)
    diff = mq - pg
    d2_col = jnp.sum(diff * diff, axis=1, keepdims=True)
    mT = jnp.where(oh_lab > 0, d2_col, -1.0)
    t5r = None
    for _ in range(_KB):
        mm = mT if t5r is None else jnp.where(mT < t5r, mT, -1.0)
        t5r = jnp.max(mm, axis=0, keepdims=True)
    t5_own = jnp.sum(oh_lab * t5r, axis=1, keepdims=True)
    sel = d2_col >= t5_own

    nc = q // _CQ
    cands = []
    for c in range(nc):
        c0 = c * _CQ
        objc = obj_ref[0, pl.ds(c0, _CQ), :]
        qnorm = jnp.sqrt(jnp.sum(objc * objc, axis=1, keepdims=True))
        allnc = objc / jnp.maximum(qnorm, 1e-12)
        a_hi, a_lo = _split2(allnc)
        simc = (_nt(a_hi, mn_hi, None) + _nt(a_hi, mn_lo, None)
                + _nt(a_lo, mn_hi, None))
        ism = jnp.any(lax.broadcasted_iota(jnp.int32, (_CQ, nm), 0)
                      == (src - c0), axis=1, keepdims=True)
        simc = jnp.where(ism, -4.0, simc)
        simbuf_ref[pl.ds(c0, _CQ), :] = simc
        tloc = None
        for _ in range(_MKNN):
            sc = simc if tloc is None else jnp.where(simc < tloc, simc, -4.0)
            tloc = jnp.max(sc, axis=0, keepdims=True)
            cands.append(tloc)

    cand = jnp.concatenate(cands, axis=0)
    t5s = None
    for _ in range(_MKNN):
        cm = cand if t5s is None else jnp.where(cand < t5s, cand, -4.0)
        t5s = jnp.max(cm, axis=0, keepdims=True)

    nv_row = jnp.zeros((1, nm), f32)
    neigh = jnp.zeros_like(mq)
    for c in range(nc):
        c0 = c * _CQ
        sc = simbuf_ref[pl.ds(c0, _CQ), :]
        wc = ((sc > _DELTA) & (sc >= t5s)).astype(jnp.bfloat16)
        nv_row = nv_row + jnp.sum(wc.astype(f32), axis=0, keepdims=True)
        o_hi, o_lo = _split2(obj_ref[0, pl.ds(c0, _CQ), :])
        neigh = neigh + _tn(wc, o_hi, None) + _tn(wc, o_lo, None)
    eye_nm = (lax.broadcasted_iota(jnp.int32, (nm, nm), 0)
              == lax.broadcasted_iota(jnp.int32, (nm, nm), 1)).astype(f32)
    nv = _nt(eye_nm, nv_row)
    g = (mq + neigh) / (1.0 + nv)

    logits = _nt(g, wcls) + bias
    mx = jnp.max(logits, axis=1, keepdims=True)
    lse = jnp.log(jnp.sum(jnp.exp(logits - mx), axis=1, keepdims=True)) + mx
    lastcol = (lax.broadcasted_iota(jnp.int32, (nm, _C), 1) == (_C - 1))
    ce = lse - jnp.sum(jnp.where(lastcol, logits, 0.0), axis=1, keepdims=True)

    valid = (sel & (nv > 0)).astype(f32)
    num = jnp.sum(ce * valid)
    den = jnp.sum(valid)

    s_t = _nt(pn, mn) / _TAU
    exps = jnp.exp(s_t)
    oh_lab_t = (lax.broadcasted_iota(jnp.int32, (_C, nm), 0) == lab
                ).astype(f32)
    cols = jnp.sum(exps, axis=1, keepdims=True)
    pos_row = jnp.sum(jnp.where(oh_lab_t > 0, s_t, 0.0), axis=0,
                      keepdims=True)
    segpos = jnp.sum(jnp.where(oh_lab_t > 0, exps, 0.0), axis=1,
                     keepdims=True)

    pos_ref[pl.ds(b, 1), :] = pos_row

    @pl.when(b == 0)
    def _():
        num_ref[...] = jnp.zeros_like(num_ref)
        den_ref[...] = jnp.zeros_like(den_ref)
        cols_ref[...] = jnp.zeros_like(cols_ref)
        segpos_ref[...] = jnp.zeros_like(segpos_ref)

    num_ref[...] += num
    den_ref[...] += den
    cols_ref[...] += cols
    segpos_ref[...] += segpos


def _k4_body(num_ref, den_ref, negp_ref, cols_ref, segpos_ref, pos_ref,
             lab_ref, out_ref):
    num = num_ref[0, 0]
    den = den_ref[0, 0]
    loss_sul = jnp.where(den > 0.0, num / jnp.maximum(den, 1.0), 0.0)

    neg = negp_ref[...] + cols_ref[...] - segpos_ref[...]
    bsz = pos_ref.shape[0]
    nm = pos_ref.shape[1]
    total = jnp.float32(0.0)
    for bi in range(bsz):
        lab_b = lab_ref[bi]
        pos_b = pos_ref[pl.ds(bi, 1), :]
        oh_t = (lax.broadcasted_iota(jnp.int32, (_C, nm), 0) == lab_b)
        negl = jnp.sum(jnp.where(oh_t, neg, 0.0), axis=0)[None, :]
        e = jnp.exp(pos_b)
        li = jnp.log((e + negl + 1e-8) / e)
        total = total + jnp.sum(li)
    loss_cec = total / jnp.float32(bsz * nm)

    lane = lax.broadcasted_iota(jnp.int32, (1, 2), 1)
    out_ref[...] = jnp.where(lane == 0, loss_sul, loss_cec)


def kernel(obj_embs, cls_means, W_cls, b_cls, src_idx, tgt_labels):
    bsz, q, d = obj_embs.shape
    nm = src_idx.shape[1]
    src3 = src_idx.reshape(bsz, 1, nm).astype(jnp.int32)
    lab3 = tgt_labels.reshape(bsz, 1, nm).astype(jnp.int32)
    labc3 = tgt_labels.reshape(bsz, nm, 1).astype(jnp.int32)
    bias2 = b_cls.reshape(1, _C)
    f32 = jnp.float32

    ntot = bsz * nm
    npad = ((ntot + 32 * _GCH - 1) // (32 * _GCH)) * 32 * _GCH
    gidx = (src_idx.astype(jnp.int32)
            + (jnp.arange(bsz, dtype=jnp.int32) * q)[:, None]).reshape(-1)
    idxp = jnp.pad(gidx, (0, npad - ntot))
    mpad = _sc_gather(obj_embs.reshape(bsz * q, d), idxp, npad, d)
    mflat = mpad[:ntot]
    matched = mflat.reshape(bsz, nm, d)
    labflat3 = tgt_labels.reshape(1, 1, ntot).astype(jnp.int32)

    protos, pn, negp = pl.pallas_call(
        _k2_body,
        out_shape=[
            jax.ShapeDtypeStruct((_C, d), f32),
            jax.ShapeDtypeStruct((_C, d), f32),
            jax.ShapeDtypeStruct((_C, 1), f32),
        ],
    )(mflat, labflat3, cls_means)

    num, den, cols, segpos, pos = pl.pallas_call(
        _k3_body,
        grid=(bsz,),
        scratch_shapes=[pltpu.VMEM((q, nm), f32)],
        in_specs=[
            pl.BlockSpec((1, q, d), lambda b: (b, 0, 0)),
            pl.BlockSpec((1, nm, d), lambda b: (b, 0, 0)),
            pl.BlockSpec((1, 1, nm), lambda b: (b, 0, 0)),
            pl.BlockSpec((1, 1, nm), lambda b: (b, 0, 0)),
            pl.BlockSpec((1, nm, 1), lambda b: (b, 0, 0)),
            pl.BlockSpec((_C, d), lambda b: (0, 0)),
            pl.BlockSpec((_C, d), lambda b: (0, 0)),
            pl.BlockSpec((_C, d), lambda b: (0, 0)),
            pl.BlockSpec((1, _C), lambda b: (0, 0)),
        ],
        out_specs=[
            pl.BlockSpec((1, 1), lambda b: (0, 0)),
            pl.BlockSpec((1, 1), lambda b: (0, 0)),
            pl.BlockSpec((_C, 1), lambda b: (0, 0)),
            pl.BlockSpec((_C, 1), lambda b: (0, 0)),
            pl.BlockSpec((bsz, nm), lambda b: (0, 0)),
        ],
        out_shape=[
            jax.ShapeDtypeStruct((1, 1), f32),
            jax.ShapeDtypeStruct((1, 1), f32),
            jax.ShapeDtypeStruct((_C, 1), f32),
            jax.ShapeDtypeStruct((_C, 1), f32),
            jax.ShapeDtypeStruct((bsz, nm), f32),
        ],
    )(obj_embs, matched, src3, lab3, labc3, protos, pn, W_cls, bias2)

    out = pl.pallas_call(
        _k4_body,
        out_shape=jax.ShapeDtypeStruct((1, 2), f32),
    )(num, den, negp, cols, segpos, pos, lab3)

    return out.reshape(2)


# speedup vs baseline: 52.6057x; 1.0271x over previous
import functools

import jax
import jax.numpy as jnp
from jax import lax
from jax.experimental import pallas as pl
from jax.experimental.pallas import tpu as pltpu
from jax.experimental.pallas import tpu_sc as plsc

_C = 81
_KNOWN = 80
_KB = 5
_MKNN = 5
_DELTA = 0.6
_TAU = 0.1
_ALPHA = 0.9
_CQ = 750
_GCH = 80
_HI = lax.Precision.HIGHEST


def _nt(a, b, precision=_HI):
    return lax.dot_general(a, b, (((1,), (1,)), ((), ())), precision=precision,
                           preferred_element_type=jnp.float32)


def _nn(a, b, precision=_HI):
    return lax.dot_general(a, b, (((1,), (0,)), ((), ())), precision=precision,
                           preferred_element_type=jnp.float32)


def _tn(a, b, precision=_HI):
    return lax.dot_general(a, b, (((0,), (0,)), ((), ())), precision=precision,
                           preferred_element_type=jnp.float32)


def _split2(x):
    hi = x.astype(jnp.bfloat16)
    lo = (x - hi.astype(jnp.float32)).astype(jnp.bfloat16)
    return hi, lo


def _normalize(x):
    n = jnp.sqrt(jnp.sum(x * x, axis=1, keepdims=True))
    return x / jnp.maximum(n, 1e-12)


def _sc_gather(table, idxp, total, d):
    nw = 32
    npw = total // (nw * _GCH)
    mesh = plsc.VectorSubcoreMesh(core_axis_name="c", subcore_axis_name="s")

    @functools.partial(
        pl.kernel, mesh=mesh,
        out_type=jax.ShapeDtypeStruct((total, d), jnp.float32),
        scratch_types=[
            pltpu.VMEM((_GCH,), jnp.int32),
            pltpu.VMEM((_GCH, d), jnp.float32),
            pltpu.SemaphoreType.DMA,
        ],
    )
    def k(table_hbm, idx_hbm, out_hbm, idx_v, rows_v, sem):
        wid = lax.axis_index("s") * 2 + lax.axis_index("c")
        for j in range(npw):
            base = pl.multiple_of((wid * npw + j) * _GCH, _GCH)
            pltpu.sync_copy(idx_hbm.at[pl.ds(base, _GCH)], idx_v)
            pltpu.async_copy(table_hbm.at[idx_v], rows_v, sem).wait()
            pltpu.sync_copy(rows_v, out_hbm.at[pl.ds(base, _GCH)])

    return k(table, idxp)


def _main_body(obj_ref, matched_ref, src_ref, lab_ref, labc_ref,
               mflat_ref, labflat_ref, cm_ref, eye_ref, labfull_ref,
               w_ref, bias_ref, out_ref,
               protos_s, pn_s, negp_s, num_s, den_s, cols_s, segpos_s,
               pos_s, simbuf_ref):
    b = pl.program_id(0)
    nb = pl.num_programs(0)
    mq = matched_ref[0]
    src = src_ref[0]
    lab = lab_ref[0]
    labc = labc_ref[0]
    wcls = w_ref[...]
    bias = bias_ref[...]
    nm, d = mq.shape
    q = obj_ref.shape[1]
    f32 = jnp.float32

    @pl.when(b == 0)
    def _():
        mflat = mflat_ref[...]
        labf = labflat_ref[0]
        cm = cm_ref[...]
        ntot = mflat.shape[0]
        oh_t = (lax.broadcasted_iota(jnp.int32, (_C, ntot), 0) == labf
                ).astype(f32)
        m_hi, m_lo = _split2(mflat)
        oh_b = oh_t.astype(jnp.bfloat16)
        seg = _nn(oh_b, m_hi, None) + _nn(oh_b, m_lo, None)
        cnt = jnp.sum(oh_t, axis=1, keepdims=True)
        means = seg / jnp.maximum(cnt, 1.0)
        upd = _normalize(_ALPHA * cm + (1.0 - _ALPHA) * means)
        protos0 = jnp.where(cnt > 0.0, upd, cm)
        protos_s[...] = protos0
        pn0 = _normalize(protos0)
        pn_s[...] = pn0
        pmat = _nt(pn0, pn0) / _TAU
        expp = jnp.exp(pmat)
        eyec = (lax.broadcasted_iota(jnp.int32, (_C, _C), 0)
                == lax.broadcasted_iota(jnp.int32, (_C, _C), 1))
        negp_s[...] = (jnp.sum(expp, axis=1, keepdims=True)
                       - jnp.sum(jnp.where(eyec, expp, 0.0), axis=1,
                                 keepdims=True))
        num_s[...] = jnp.zeros_like(num_s)
        den_s[...] = jnp.zeros_like(den_s)
        cols_s[...] = jnp.zeros_like(cols_s)
        segpos_s[...] = jnp.zeros_like(segpos_s)

    protos = protos_s[...]
    pn = pn_s[...]

    mnorm = jnp.sqrt(jnp.sum(mq * mq, axis=1, keepdims=True))
    mn = mq / jnp.maximum(mnorm, 1e-12)
    mn_hi, mn_lo = _split2(mn)

    oh_lab = (lax.broadcasted_iota(jnp.int32, (nm, _C), 1) == labc
              ).astype(f32)
    pg = _nn(oh_lab, protos)
    diff = mq - pg
    d2_col = jnp.sum(diff * diff, axis=1, keepdims=True)
    mT = jnp.where(oh_lab > 0, d2_col, -1.0)
    t5r = None
    for _ in range(_KB):
        mm = mT if t5r is None else jnp.where(mT < t5r, mT, -1.0)
        t5r = jnp.max(mm, axis=0, keepdims=True)
    t5_own = jnp.sum(oh_lab * t5r, axis=1, keepdims=True)
    sel = d2_col >= t5_own

    nc = q // _CQ
    cands = []
    for c in range(nc):
        c0 = c * _CQ
        objc = obj_ref[0, pl.ds(c0, _CQ), :]
        qnorm = jnp.sqrt(jnp.sum(objc * objc, axis=1, keepdims=True))
        allnc = objc / jnp.maximum(qnorm, 1e-12)
        a_hi, a_lo = _split2(allnc)
        simc = (_nt(a_hi, mn_hi, None) + _nt(a_hi, mn_lo, None)
                + _nt(a_lo, mn_hi, None))
        ism = jnp.any(lax.broadcasted_iota(jnp.int32, (_CQ, nm), 0)
                      == (src - c0), axis=1, keepdims=True)
        simc = jnp.where(ism, -4.0, simc)
        simbuf_ref[pl.ds(c0, _CQ), :] = simc
        tloc = None
        for _ in range(_MKNN):
            sc = simc if tloc is None else jnp.where(simc < tloc, simc, -4.0)
            tloc = jnp.max(sc, axis=0, keepdims=True)
            cands.append(tloc)

    cand = jnp.concatenate(cands, axis=0)
    t5s = None
    for _ in range(_MKNN):
        cm2 = cand if t5s is None else jnp.where(cand < t5s, cand, -4.0)
        t5s = jnp.max(cm2, axis=0, keepdims=True)

    nv_row = jnp.zeros((1, nm), f32)
    neigh = jnp.zeros_like(mq)
    for c in range(nc):
        c0 = c * _CQ
        sc = simbuf_ref[pl.ds(c0, _CQ), :]
        wcb = (sc > _DELTA) & (sc >= t5s)
        nv_row = nv_row + jnp.sum(wcb.astype(f32), axis=0, keepdims=True)
        wc = wcb.astype(jnp.bfloat16)
        o_hi, o_lo = _split2(obj_ref[0, pl.ds(c0, _CQ), :])
        neigh = neigh + _tn(wc, o_hi, None) + _tn(wc, o_lo, None)
    nv = _nt(eye_ref[...], nv_row)
    g = (mq + neigh) / (1.0 + nv)

    logits = _nt(g, wcls) + bias
    mx = jnp.max(logits, axis=1, keepdims=True)
    lse = jnp.log(jnp.sum(jnp.exp(logits - mx), axis=1, keepdims=True)) + mx
    lastcol = (lax.broadcasted_iota(jnp.int32, (nm, _C), 1) == (_C - 1))
    ce = lse - jnp.sum(jnp.where(lastcol, logits, 0.0), axis=1, keepdims=True)

    valid = (sel & (nv > 0)).astype(f32)

    s_t = _nt(pn, mn) / _TAU
    exps = jnp.exp(s_t)
    oh_lab_t = (lax.broadcasted_iota(jnp.int32, (_C, nm), 0) == lab
                ).astype(f32)
    pos_row = jnp.sum(jnp.where(oh_lab_t > 0, s_t, 0.0), axis=0,
                      keepdims=True)
    pos_s[pl.ds(b, 1), :] = pos_row

    num_s[...] += jnp.sum(ce * valid)
    den_s[...] += jnp.sum(valid)
    cols_s[...] += jnp.sum(exps, axis=1, keepdims=True)
    segpos_s[...] += jnp.sum(jnp.where(oh_lab_t > 0, exps, 0.0), axis=1,
                             keepdims=True)

    @pl.when(b == nb - 1)
    def _():
        num = num_s[0, 0]
        den = den_s[0, 0]
        loss_sul = jnp.where(den > 0.0, num / jnp.maximum(den, 1.0), 0.0)
        neg = negp_s[...] + cols_s[...] - segpos_s[...]
        total = jnp.float32(0.0)
        for bi in range(nb):
            lab_b = labfull_ref[bi]
            pos_b = pos_s[pl.ds(bi, 1), :]
            oh2 = (lax.broadcasted_iota(jnp.int32, (_C, nm), 0) == lab_b)
            negl = jnp.sum(jnp.where(oh2, neg, 0.0), axis=0)[None, :]
            e = jnp.exp(pos_b)
            li = jnp.log((e + negl + 1e-8) / e)
            total = total + jnp.sum(li)
        loss_cec = total / jnp.float32(nb * nm)
        lane = lax.broadcasted_iota(jnp.int32, (1, 2), 1)
        out_ref[...] = jnp.where(lane == 0, loss_sul, loss_cec)


def kernel(obj_embs, cls_means, W_cls, b_cls, src_idx, tgt_labels):
    bsz, q, d = obj_embs.shape
    nm = src_idx.shape[1]
    src3 = src_idx.reshape(bsz, 1, nm).astype(jnp.int32)
    lab3 = tgt_labels.reshape(bsz, 1, nm).astype(jnp.int32)
    labc3 = tgt_labels.reshape(bsz, nm, 1).astype(jnp.int32)
    labflat3 = tgt_labels.reshape(1, 1, bsz * nm).astype(jnp.int32)
    bias2 = b_cls.reshape(1, _C)
    eye_nm = jnp.eye(nm, dtype=jnp.float32)
    f32 = jnp.float32
    ntot = bsz * nm

    npad = ((ntot + 32 * _GCH - 1) // (32 * _GCH)) * 32 * _GCH
    gidx = (src_idx.astype(jnp.int32)
            + (jnp.arange(bsz, dtype=jnp.int32) * q)[:, None]).reshape(-1)
    idxp = jnp.pad(gidx, (0, npad - ntot))
    mpad = _sc_gather(obj_embs.reshape(bsz * q, d), idxp, npad, d)
    mflat = mpad[:ntot]
    matched = mflat.reshape(bsz, nm, d)

    out = pl.pallas_call(
        _main_body,
        grid=(bsz,),
        in_specs=[
            pl.BlockSpec((1, q, d), lambda b: (b, 0, 0)),
            pl.BlockSpec((1, nm, d), lambda b: (b, 0, 0)),
            pl.BlockSpec((1, 1, nm), lambda b: (b, 0, 0)),
            pl.BlockSpec((1, 1, nm), lambda b: (b, 0, 0)),
            pl.BlockSpec((1, nm, 1), lambda b: (b, 0, 0)),
            pl.BlockSpec((ntot, d), lambda b: (0, 0)),
            pl.BlockSpec((1, 1, ntot), lambda b: (0, 0, 0)),
            pl.BlockSpec((_C, d), lambda b: (0, 0)),
            pl.BlockSpec((nm, nm), lambda b: (0, 0)),
            pl.BlockSpec((bsz, 1, nm), lambda b: (0, 0, 0)),
            pl.BlockSpec((_C, d), lambda b: (0, 0)),
            pl.BlockSpec((1, _C), lambda b: (0, 0)),
        ],
        out_specs=pl.BlockSpec((1, 2), lambda b: (0, 0)),
        out_shape=jax.ShapeDtypeStruct((1, 2), f32),
        scratch_shapes=[
            pltpu.VMEM((_C, d), f32),
            pltpu.VMEM((_C, d), f32),
            pltpu.VMEM((_C, 1), f32),
            pltpu.VMEM((1, 1), f32),
            pltpu.VMEM((1, 1), f32),
            pltpu.VMEM((_C, 1), f32),
            pltpu.VMEM((_C, 1), f32),
            pltpu.VMEM((bsz, nm), f32),
            pltpu.VMEM((q, nm), f32),
        ],
    )(obj_embs, matched, src3, lab3, labc3, mflat, labflat3, cls_means,
      eye_nm, lab3, W_cls, bias2)

    return out.reshape(2)
